# baseline (speedup 1.0000x reference)
import functools

import jax
import jax.numpy as jnp
from jax import lax
from jax.experimental import pallas as pl
from jax.experimental.pallas import tpu as pltpu
from jax.experimental.pallas import tpu_sc as plsc

_D = 64
_BATCH = 4096
_SEQ = 200
_B_TOTAL = _BATCH * _SEQ
_NC = 2
_NS = 16
_NW = _NC * _NS
_B_PER_W = _B_TOTAL // _NW
_CHUNK = 200
_NBUF = 2
_N_ITERS = _B_PER_W // _CHUNK
_N_GROUPS = _N_ITERS // _NBUF


def _gather_body(table_hbm, idx_hbm, out_hbm, idx_v, buf0, buf1, sg0, sg1,
                 sw0, sw1):
    wid = lax.axis_index("s") * _NC + lax.axis_index("c")
    base0 = wid * _B_PER_W
    bufs = (buf0, buf1)
    sgs = (sg0, sg1)
    sws = (sw0, sw1)

    def start_gather(i, b):
        pltpu.async_copy(
            table_hbm.at[idx_v.at[pl.ds(i * _CHUNK, _CHUNK)]], bufs[b],
            sgs[b])

    def wait_gather(b):
        pltpu.make_async_copy(
            table_hbm.at[idx_v.at[pl.ds(0, _CHUNK)]], bufs[b], sgs[b]).wait()

    def start_write(i, b):
        dst = out_hbm.at[pl.ds(base0 + i * _CHUNK, _CHUNK)]
        pltpu.async_copy(bufs[b], dst, sws[b])

    def wait_write(b):
        dst = out_hbm.at[pl.ds(base0, _CHUNK)]
        pltpu.make_async_copy(bufs[b], dst, sws[b]).wait()

    pltpu.sync_copy(idx_hbm.at[wid], idx_v)

    start_gather(0, 0)
    start_gather(1, 1)

    def group(g, carry):
        for b in range(_NBUF):
            i = g * _NBUF + b
            wait_gather(b)
            start_write(i, b)

        @pl.when(g < _N_GROUPS - 1)
        def _():
            for b in range(_NBUF):
                i = g * _NBUF + b
                wait_write(b)
                start_gather(i + _NBUF, b)

        return carry

    lax.fori_loop(0, _N_GROUPS, group, 0)

    for b in range(_NBUF):
        wait_write(b)


@jax.jit
def kernel(x, vocab_table):
    mesh = plsc.VectorSubcoreMesh(core_axis_name="c", subcore_axis_name="s")
    gather = functools.partial(
        pl.kernel,
        mesh=mesh,
        out_type=jax.ShapeDtypeStruct((_B_TOTAL, 2 * _D), jnp.float32),
        scratch_types=[
            pltpu.VMEM((_B_PER_W,), jnp.int32),
            pltpu.VMEM((_CHUNK, 2 * _D), jnp.float32),
            pltpu.VMEM((_CHUNK, 2 * _D), jnp.float32),
            pltpu.SemaphoreType.DMA,
            pltpu.SemaphoreType.DMA,
            pltpu.SemaphoreType.DMA,
            pltpu.SemaphoreType.DMA,
        ],
        compiler_params=pltpu.CompilerParams(use_tc_tiling_on_sc=False),
    )(_gather_body)
    table_pairs = vocab_table.reshape(500000, 2 * _D)
    pair_idx = (x >> 1).reshape(_NW, _B_PER_W)
    rows = gather(table_pairs, pair_idx)
    rows3 = rows.reshape(_BATCH, _SEQ, 2 * _D)
    odd = (x & 1).astype(bool)[:, :, None]
    return jnp.where(odd, rows3[:, :, _D:], rows3[:, :, :_D])


# speedup vs baseline: 2.1491x; 2.1491x over previous
import functools

import jax
import jax.numpy as jnp
from jax import lax
from jax.experimental import pallas as pl
from jax.experimental.pallas import tpu as pltpu
from jax.experimental.pallas import tpu_sc as plsc

_D = 64
_L = 16
_BATCH = 4096
_SEQ = 200
_B_TOTAL = _BATCH * _SEQ
_NC = 2
_NS = 16
_NW = _NC * _NS
_B_PER_W = _B_TOTAL // _NW
_CHUNK = 256
_HALF = _CHUNK // 2
_N_ITERS = _B_PER_W // _CHUNK
_N_PAIRS = _N_ITERS // 2


def _gather_body(table_hbm, idx_hbm, out_hbm, idx_v, bufg0, bufg1,
                 bufw0, bufw1, sg0, sg1, sw0, sw1):
    wid = lax.axis_index("s") * _NC + lax.axis_index("c")
    base0 = wid * _B_PER_W
    bufgs = (bufg0, bufg1)
    bufws = (bufw0, bufw1)
    sgs = (sg0, sg1)
    sws = (sw0, sw1)

    def start_gather(i, b):
        pltpu.async_copy(
            table_hbm.at[idx_v.at[pl.ds(i * _CHUNK, _CHUNK)]], bufgs[b],
            sgs[b])

    def wait_gather(b):
        pltpu.make_async_copy(
            table_hbm.at[idx_v.at[pl.ds(0, _CHUNK)]], bufgs[b], sgs[b]).wait()

    def start_write(i, h):
        dst = out_hbm.at[pl.ds(base0 + i * _CHUNK + h * _HALF, _HALF)]
        pltpu.async_copy(bufws[h], dst, sws[h])

    def wait_write(h):
        dst = out_hbm.at[pl.ds(base0, _HALF)]
        pltpu.make_async_copy(bufws[h], dst, sws[h]).wait()

    def extract_half(b, h):
        src = bufgs[b]
        dst = bufws[h]

        def ebody(it, carry):
            r0 = it * _L
            for u in range(_L):
                r = r0 + u
                for k in range(_D // _L):
                    dst[r, pl.ds(k * _L, _L)] = (
                        src[h * _HALF + r, pl.ds(k * _L, _L)])
            return carry

        lax.fori_loop(0, _HALF // _L, ebody, 0)

    pltpu.sync_copy(idx_hbm.at[wid], idx_v)

    start_gather(0, 0)
    start_gather(1, 1)

    def pair(p, carry):
        for b in range(2):
            i = 2 * p + b
            wait_gather(b)
            for h in range(2):
                if b == 0:
                    @pl.when(p > 0)
                    def _():
                        wait_write(h)
                else:
                    wait_write(h)
                extract_half(b, h)
                start_write(i, h)

            @pl.when(p < _N_PAIRS - 1)
            def _():
                start_gather(i + 2, b)

        return carry

    lax.fori_loop(0, _N_PAIRS, pair, 0)

    for h in range(2):
        wait_write(h)


@jax.jit
def kernel(x, vocab_table):
    mesh = plsc.VectorSubcoreMesh(core_axis_name="c", subcore_axis_name="s")
    gather = functools.partial(
        pl.kernel,
        mesh=mesh,
        out_type=jax.ShapeDtypeStruct((_B_TOTAL, _D), jnp.float32),
        scratch_types=[
            pltpu.VMEM((_B_PER_W,), jnp.int32),
            pltpu.VMEM((_CHUNK, 2 * _D), jnp.float32),
            pltpu.VMEM((_CHUNK, 2 * _D), jnp.float32),
            pltpu.VMEM((_HALF, _D), jnp.float32),
            pltpu.VMEM((_HALF, _D), jnp.float32),
            pltpu.SemaphoreType.DMA,
            pltpu.SemaphoreType.DMA,
            pltpu.SemaphoreType.DMA,
            pltpu.SemaphoreType.DMA,
        ],
        compiler_params=pltpu.CompilerParams(use_tc_tiling_on_sc=True),
    )(_gather_body)
    table_wide = jnp.pad(vocab_table, ((0, 0), (0, _D)))
    out = gather(table_wide, x.reshape(_NW, _B_PER_W))
    return out.reshape(_BATCH, _SEQ, _D)
